# Initial kernel scaffold; baseline (speedup 1.0000x reference)
import jax
import jax.numpy as jnp
from jax.experimental import pallas as pl


def kernel(xs, table):
    raise NotImplementedError("write your pallas kernel here")


# speedup vs baseline: 2.2108x; 2.2108x over previous
import functools

import jax
import jax.numpy as jnp
from jax import lax
from jax.experimental import pallas as pl
from jax.experimental.pallas import tpu as pltpu
from jax.experimental.pallas import tpu_sc as plsc

VOCAB = 1000000
EMB_D = 32
BATCH = 4096
HIST = 200

_info = plsc.get_sparse_core_info()
NC = _info.num_cores
NS = _info.num_subcores
LANES = _info.num_lanes
NW = NC * NS
BPW = BATCH // NW
SC_ROWS = NS * BPW


def _make_kernel():
    mesh = plsc.VectorSubcoreMesh(core_axis_name="c", subcore_axis_name="s")

    @functools.partial(
        pl.kernel,
        mesh=mesh,
        out_type=jax.ShapeDtypeStruct((BATCH, EMB_D), jnp.float32),
        compiler_params=pltpu.CompilerParams(use_tc_tiling_on_sc=False),
        scratch_types=[
            pltpu.VMEM((HIST, BPW), jnp.int32),
            pltpu.VMEM((BPW,), jnp.int32),
            pltpu.VMEM((BPW, EMB_D), jnp.float32),
            pltpu.VMEM((BPW, EMB_D), jnp.float32),
            pltpu.VMEM_SHARED((SC_ROWS, EMB_D), jnp.float32),
            pltpu.SemaphoreType.DMA,
            pltpu.SemaphoreType.DMA,
        ],
    )
    def k(xst_hbm, dpat_hbm, table_hbm, out_hbm,
          idx_v, dst_v, buf_a, buf_b, acc, sem_a, sem_b):
        c = lax.axis_index("c")
        s = lax.axis_index("s")
        w = c * NS + s

        pltpu.sync_copy(xst_hbm.at[w], idx_v)
        pltpu.sync_copy(dpat_hbm.at[s], dst_v)

        def g_start(l, buf, sem):
            pltpu.async_copy(table_hbm.at[idx_v.at[l]], buf, sem)

        def g_wait(l, buf, sem):
            pltpu.make_async_copy(table_hbm.at[idx_v.at[l]], buf, sem).wait()

        g_start(0, buf_a, sem_a)
        g_start(1, buf_b, sem_b)
        g_wait(0, buf_a, sem_a)
        pltpu.sync_copy(buf_a, acc.at[pl.ds(s * BPW, BPW)])
        g_start(2, buf_a, sem_a)
        g_wait(1, buf_b, sem_b)
        pltpu.sync_copy(buf_b, acc.at[dst_v], add=True)
        g_start(3, buf_b, sem_b)

        def body(kk, carry):
            la = 2 * kk
            g_wait(la, buf_a, sem_a)
            pltpu.sync_copy(buf_a, acc.at[dst_v], add=True)
            g_start(la + 2, buf_a, sem_a)
            g_wait(la + 1, buf_b, sem_b)
            pltpu.sync_copy(buf_b, acc.at[dst_v], add=True)
            g_start(la + 3, buf_b, sem_b)
            return carry

        lax.fori_loop(1, HIST // 2 - 1, body, 0)

        g_wait(HIST - 2, buf_a, sem_a)
        pltpu.sync_copy(buf_a, acc.at[dst_v], add=True)
        g_wait(HIST - 1, buf_b, sem_b)
        pltpu.sync_copy(buf_b, acc.at[dst_v], add=True)

        pltpu.sync_copy(acc.at[pl.ds(s * BPW, BPW)], buf_a)
        inv = jnp.float32(1.0 / HIST)

        def sbody(j, carry):
            buf_a[j, pl.ds(0, LANES)] = buf_a[j, pl.ds(0, LANES)] * inv
            buf_a[j, pl.ds(LANES, LANES)] = buf_a[j, pl.ds(LANES, LANES)] * inv
            return carry

        lax.fori_loop(0, BPW, sbody, 0)
        pltpu.sync_copy(buf_a, out_hbm.at[pl.ds(w * BPW, BPW)])

    return k


_sc_kernel = _make_kernel()


def kernel(xs, table):
    xs32 = xs.astype(jnp.int32)
    xst = xs32.reshape(NW, BPW, HIST).transpose(0, 2, 1)
    dpat = jnp.arange(SC_ROWS, dtype=jnp.int32).reshape(NS, BPW)
    return _sc_kernel(xst, dpat, table)
